# baseline (speedup 1.0000x reference)
import functools

import jax
import jax.numpy as jnp
from jax import lax
from jax.experimental import pallas as pl
from jax.experimental.pallas import tpu as pltpu
from jax.experimental.pallas import tpu_sc as plsc

_VOCAB = 1000000
_EMB = 64
_PAD = 128
_NEG = 10
_BATCH = 16384

_NC = 2
_NS = 16
_NW = _NC * _NS
_CH = 128


def _noise_flat():
    nz = jax.random.randint(jax.random.key(42), (_BATCH, _NEG), 0, _VOCAB)
    return nz.astype(jnp.int32).reshape(-1)


def _sc_gather(input_word, context_word, noise_flat, W_in, W_ctx):
    B = _BATCH
    NB = _BATCH * _NEG
    mesh = plsc.VectorSubcoreMesh(core_axis_name="c", subcore_axis_name="s")
    out_types = (
        jax.ShapeDtypeStruct((B, _PAD), jnp.float32),
        jax.ShapeDtypeStruct((B, _PAD), jnp.float32),
        jax.ShapeDtypeStruct((NB, _PAD), jnp.float32),
    )

    b_w = B // _NW
    n_w = NB // _NW
    nneg = n_w // _CH

    @functools.partial(
        pl.kernel,
        mesh=mesh,
        out_type=out_types,
        compiler_params=pltpu.CompilerParams(use_tc_tiling_on_sc=False),
        scratch_types=[
            pltpu.VMEM((_CH,), jnp.int32),
            pltpu.VMEM((_CH,), jnp.int32),
            pltpu.VMEM((_CH, _EMB), jnp.float32),
            pltpu.VMEM((_CH, _EMB), jnp.float32),
            pltpu.SemaphoreType.DMA,
            pltpu.SemaphoreType.DMA,
        ],
    )
    def k(iw_hbm, cw_hbm, nz_hbm, win_hbm, wctx_hbm,
          oin_hbm, octx_hbm, oneg_hbm, idxc0, idxc1,
          rows0, rows1, sem0, sem1):
        wid = lax.axis_index("s") * _NC + lax.axis_index("c")

        bufs, sems = (rows0, rows1), (sem0, sem1)
        idxcs = (idxc0, idxc1)

        def gather_phase(ihbm, ibase, table, out, obase, nch):
            @pl.loop(0, nch // 2)
            def _(t):
                offs = [ibase + (2 * t) * _CH, ibase + (2 * t + 1) * _CH]
                oofs = [obase + (2 * t) * _CH, obase + (2 * t + 1) * _CH]
                cps = []
                for b in (0, 1):
                    pltpu.sync_copy(ihbm.at[pl.ds(offs[b], _CH)], idxcs[b])
                    cps.append(pltpu.async_copy(
                        table.at[idxcs[b]], bufs[b], sems[b]))
                for b in (0, 1):
                    cps[b].wait()
                    pltpu.sync_copy(
                        bufs[b], out.at[pl.ds(oofs[b], _CH), pl.ds(0, _EMB)])

        gather_phase(iw_hbm, wid * b_w, win_hbm, oin_hbm, wid * b_w,
                     b_w // _CH)
        gather_phase(cw_hbm, wid * b_w, wctx_hbm, octx_hbm, wid * b_w,
                     b_w // _CH)
        gather_phase(nz_hbm, wid * n_w, wctx_hbm, oneg_hbm, wid * n_w,
                     n_w // _CH)

    return k(input_word, context_word, noise_flat, W_in, W_ctx)


def _tc_loss(emb_in, emb_ctx, emb_neg):
    B = _BATCH
    Bb = 1024
    G = B // Bb

    def body(in_ref, ctx_ref, neg_ref, acc_ref):
        a = in_ref[:, : _EMB]
        c = ctx_ref[:, : _EMB]
        n = neg_ref[:, : _EMB].reshape(Bb, _NEG, _EMB)
        pos = jnp.sum(a * c, axis=1, keepdims=True)
        negs = jnp.sum(n * a[:, None, :], axis=2)

        def logsig(x):
            return jnp.minimum(x, 0.0) - jnp.log1p(jnp.exp(-jnp.abs(x)))

        total = jnp.sum(logsig(pos)) + jnp.sum(logsig(-negs))

        @pl.when(pl.program_id(0) == 0)
        def _():
            acc_ref[...] = jnp.zeros((1, 1), jnp.float32)

        acc_ref[...] += jnp.reshape(total, (1, 1))

    acc = pl.pallas_call(
        body,
        grid=(G,),
        in_specs=[
            pl.BlockSpec((Bb, _PAD), lambda i: (i, 0)),
            pl.BlockSpec((Bb, _PAD), lambda i: (i, 0)),
            pl.BlockSpec((Bb * _NEG, _PAD), lambda i: (i, 0)),
        ],
        out_specs=pl.BlockSpec((1, 1), lambda i: (0, 0)),
        out_shape=jax.ShapeDtypeStruct((1, 1), jnp.float32),
    )(emb_in, emb_ctx, emb_neg)
    return -acc[0, 0] / B


def kernel(input_word, context_word, W_in, W_ctx):
    iw = input_word.astype(jnp.int32)
    cw = context_word.astype(jnp.int32)
    nz = _noise_flat()
    emb_in, emb_ctx, emb_neg = _sc_gather(iw, cw, nz, W_in, W_ctx)
    return _tc_loss(emb_in, emb_ctx, emb_neg)
